# Initial kernel scaffold; baseline (speedup 1.0000x reference)
import jax
import jax.numpy as jnp
from jax.experimental import pallas as pl


def kernel(x, edge_features, Wm1, bm1, Wm2, bm2, Wu1, bu1, Wu2, bu2, edge_index):
    raise NotImplementedError("write your pallas kernel here")


# speedup vs baseline: 3.2991x; 3.2991x over previous
import functools

import jax
import jax.numpy as jnp
from jax import lax
from jax.experimental import pallas as pl
from jax.experimental.pallas import tpu as pltpu
from jax.experimental.pallas import tpu_sc as plsc

NC = 2
NS = 16
LANES = 16
CH = 80


def _pq_kernel(x_ref, wa_ref, wb_ref, p_ref, q_ref):
    xb = x_ref[...]
    p_ref[...] = jnp.dot(xb, wa_ref[...], preferred_element_type=jnp.float32)
    q_ref[...] = jnp.dot(xb, wb_ref[...], preferred_element_type=jnp.float32)


def _r_kernel(ef_ref, we_ref, b_ref, r_ref):
    r_ref[...] = (
        jnp.dot(jnp.log1p(ef_ref[...]), we_ref[...],
                preferred_element_type=jnp.float32)
        + b_ref[...]
    )


def _fin_kernel(sp_ref, x_ref, wm2_ref, bm2_ref, wu1a_ref, wu1b_ref,
                bu1_ref, wu2_ref, bu2_ref, o_ref, *, hm):
    s = sp_ref[0, :, :hm] + sp_ref[1, :, :hm]
    deg = sp_ref[0, :, hm:hm + 1] + sp_ref[1, :, hm:hm + 1]
    agg = (jnp.dot(s, wm2_ref[...], preferred_element_type=jnp.float32)
           + deg * bm2_ref[...])
    h2 = jnp.maximum(
        jnp.dot(x_ref[...], wu1a_ref[...], preferred_element_type=jnp.float32)
        + jnp.dot(agg, wu1b_ref[...], preferred_element_type=jnp.float32)
        + bu1_ref[...], 0.0)
    o_ref[...] = (jnp.dot(h2, wu2_ref[...], preferred_element_type=jnp.float32)
                  + bu2_ref[...])


def _edge_sc(src2, dst2, p_tab, q_tab, r_tab, *, n_nodes, hm, e_total):
    sw = hm + LANES
    nw = NC * NS
    ept = e_total // nw
    k_chunks = ept // CH
    rt = ((n_nodes + NS - 1) // NS + CH - 1) // CH * CH
    npad = rt * NS
    nvr = hm // LANES

    mesh = plsc.VectorSubcoreMesh(core_axis_name="c", subcore_axis_name="s",
                                  num_cores=NC, num_subcores=NS)

    @functools.partial(
        pl.kernel, mesh=mesh,
        compiler_params=pltpu.CompilerParams(use_tc_tiling_on_sc=False),
        out_type=jax.ShapeDtypeStruct((NC * npad, sw), jnp.float32),
        scratch_types=[
            pltpu.VMEM((k_chunks, CH), jnp.int32),
            pltpu.VMEM((k_chunks, CH), jnp.int32),
            pltpu.VMEM((CH, hm), jnp.float32),
            pltpu.VMEM((CH, hm), jnp.float32),
            pltpu.VMEM((CH, hm), jnp.float32),
            pltpu.VMEM((CH, sw), jnp.float32),
            pltpu.VMEM_SHARED((npad, sw), jnp.float32),
            pltpu.SemaphoreType.DMA,
            pltpu.SemaphoreType.DMA,
            pltpu.SemaphoreType.DMA,
        ],
    )
    def edge_kernel(src_hbm, dst_hbm, p_hbm, q_hbm, r_hbm, out_hbm,
                    src_v, dst_v, p_v, q_v, r_v, h_v, s_acc,
                    sem_p, sem_q, sem_r):
        c = lax.axis_index("c")
        s = lax.axis_index("s")
        wid = c * NS + s

        zero = jnp.zeros((LANES,), jnp.float32)

        def zero_row(i, _):
            for j in range(sw // LANES):
                h_v[i, pl.ds(j * LANES, LANES)] = zero
            return 0

        lax.fori_loop(0, CH, zero_row, 0)

        for t in range(rt // CH):
            pltpu.sync_copy(h_v, s_acc.at[pl.ds(s * rt + t * CH, CH)])

        onehot = jnp.where(lax.iota(jnp.int32, LANES) == 0, 1.0, 0.0)

        def one_row(i, _):
            h_v[i, pl.ds(hm, LANES)] = onehot
            return 0

        lax.fori_loop(0, CH, one_row, 0)

        pltpu.sync_copy(src_hbm.at[wid], src_v)
        pltpu.sync_copy(dst_hbm.at[wid], dst_v)

        plsc.subcore_barrier()

        def chunk(j, _):
            base = wid * ept + j * CH
            cp = pltpu.async_copy(p_hbm.at[src_v.at[j]], p_v, sem_p)
            cq = pltpu.async_copy(q_hbm.at[dst_v.at[j]], q_v, sem_q)
            cr = pltpu.async_copy(r_hbm.at[pl.ds(base, CH)], r_v, sem_r)
            cp.wait()
            cq.wait()
            cr.wait()

            def edge(i, _):
                for v in range(nvr):
                    sl = pl.ds(v * LANES, LANES)
                    h_v[i, sl] = jnp.maximum(
                        p_v[i, sl] + q_v[i, sl] + r_v[i, sl], 0.0)
                return 0

            lax.fori_loop(0, CH, edge, 0)
            pltpu.sync_copy(h_v, s_acc.at[dst_v.at[j]], add=True)
            return 0

        lax.fori_loop(0, k_chunks, chunk, 0)

        plsc.subcore_barrier()

        pltpu.sync_copy(s_acc.at[pl.ds(s * rt, rt)],
                        out_hbm.at[pl.ds(c * npad + s * rt, rt)])

    return edge_kernel(src2, dst2, p_tab, q_tab, r_tab), npad, sw


def kernel(x, edge_features, Wm1, bm1, Wm2, bm2, Wu1, bu1, Wu2, bu2, edge_index):
    n, d = x.shape
    e, fe = edge_features.shape
    hm = Wm1.shape[1]
    dout = Wu2.shape[1]

    assert e % (NC * NS * CH) == 0 and hm % LANES == 0

    bn = 2000
    assert n % bn == 0
    p_tab, q_tab = pl.pallas_call(
        _pq_kernel,
        grid=(n // bn,),
        in_specs=[
            pl.BlockSpec((bn, d), lambda i: (i, 0)),
            pl.BlockSpec((d, hm), lambda i: (0, 0)),
            pl.BlockSpec((d, hm), lambda i: (0, 0)),
        ],
        out_specs=[
            pl.BlockSpec((bn, hm), lambda i: (i, 0)),
            pl.BlockSpec((bn, hm), lambda i: (i, 0)),
        ],
        out_shape=[
            jax.ShapeDtypeStruct((n, hm), jnp.float32),
            jax.ShapeDtypeStruct((n, hm), jnp.float32),
        ],
    )(x, Wm1[:d], Wm1[d:2 * d])

    be = 3200
    assert e % be == 0
    r_tab = pl.pallas_call(
        _r_kernel,
        grid=(e // be,),
        in_specs=[
            pl.BlockSpec((be, fe), lambda i: (i, 0)),
            pl.BlockSpec((fe, hm), lambda i: (0, 0)),
            pl.BlockSpec((1, hm), lambda i: (0, 0)),
        ],
        out_specs=pl.BlockSpec((be, hm), lambda i: (i, 0)),
        out_shape=jax.ShapeDtypeStruct((e, hm), jnp.float32),
    )(edge_features, Wm1[2 * d:], bm1.reshape(1, hm))

    nw = NC * NS
    src2 = edge_index[0].reshape(nw, e // (nw * CH), CH)
    dst2 = edge_index[1].reshape(nw, e // (nw * CH), CH)
    sp, npad, sw = _edge_sc(src2, dst2, p_tab, q_tab, r_tab,
                            n_nodes=n, hm=hm, e_total=e)
    sp = sp.reshape(NC, npad, sw)

    out = pl.pallas_call(
        functools.partial(_fin_kernel, hm=hm),
        grid=(n // bn,),
        in_specs=[
            pl.BlockSpec((NC, bn, sw), lambda i: (0, i, 0)),
            pl.BlockSpec((bn, d), lambda i: (i, 0)),
            pl.BlockSpec((hm, dout), lambda i: (0, 0)),
            pl.BlockSpec((1, dout), lambda i: (0, 0)),
            pl.BlockSpec((d, Wu1.shape[1]), lambda i: (0, 0)),
            pl.BlockSpec((d, Wu1.shape[1]), lambda i: (0, 0)),
            pl.BlockSpec((1, Wu1.shape[1]), lambda i: (0, 0)),
            pl.BlockSpec((Wu1.shape[1], dout), lambda i: (0, 0)),
            pl.BlockSpec((1, dout), lambda i: (0, 0)),
        ],
        out_specs=pl.BlockSpec((bn, dout), lambda i: (i, 0)),
        out_shape=jax.ShapeDtypeStruct((n, dout), jnp.float32),
    )(sp, x, Wm2, bm2.reshape(1, dout), Wu1[:d], Wu1[d:],
      bu1.reshape(1, -1), Wu2, bu2.reshape(1, dout))
    return out
